# baseline (speedup 1.0000x reference)
import functools

import jax
import jax.numpy as jnp
from jax import lax
from jax.experimental import pallas as pl
from jax.experimental.pallas import tpu as pltpu
from jax.experimental.pallas import tpu_sc as plsc

E = 160000
D = 256
NW = 32
K = 80
MAIN_STEPS = 62
MAIN_E = NW * MAIN_STEPS * K
IDX_CAP = (MAIN_STEPS + 1) * K
LANES = 16
GROUPS = K // LANES


def _dot_decoder_body(src_hbm, dst_hbm, zsrc_hbm, zdst_hbm, out_hbm,
                      idx_s, idx_d, rows_s, rows_d, out_v, sem_s, sem_d):
    wid = lax.axis_index("s") * 2 + lax.axis_index("c")
    wbase = wid * MAIN_STEPS * K
    tail_base = MAIN_E + wid * K

    pltpu.sync_copy(src_hbm.at[pl.ds(wbase, MAIN_STEPS * K)],
                    idx_s.at[pl.ds(0, MAIN_STEPS * K)])
    pltpu.sync_copy(dst_hbm.at[pl.ds(wbase, MAIN_STEPS * K)],
                    idx_d.at[pl.ds(0, MAIN_STEPS * K)])

    @pl.when(wid < 16)
    def _stage_tail():
        pltpu.sync_copy(src_hbm.at[pl.ds(tail_base, K)],
                        idx_s.at[pl.ds(MAIN_STEPS * K, K)])
        pltpu.sync_copy(dst_hbm.at[pl.ds(tail_base, K)],
                        idx_d.at[pl.ds(MAIN_STEPS * K, K)])

    nsteps = jnp.where(wid < 16, MAIN_STEPS + 1, MAIN_STEPS)

    def step(i, _):
        cp_s = pltpu.async_copy(
            zsrc_hbm.at[idx_s.at[pl.ds(i * K, K)]], rows_s, sem_s)
        cp_d = pltpu.async_copy(
            zdst_hbm.at[idx_d.at[pl.ds(i * K, K)]], rows_d, sem_d)
        cp_s.wait()
        cp_d.wait()

        for g in range(GROUPS):
            rowids = jnp.arange(LANES, dtype=jnp.int32) + g * LANES

            def col(c, acc):
                cvec = jnp.full((LANES,), c, dtype=jnp.int32)
                a = plsc.load_gather(rows_s, [rowids, cvec])
                b = plsc.load_gather(rows_d, [rowids, cvec])
                return acc + a * b

            out16 = lax.fori_loop(0, D, col, jnp.zeros((LANES,), jnp.float32),
                                  unroll=8)
            out_v[pl.ds(g * LANES, LANES)] = out16

        obase = jnp.where(i < MAIN_STEPS, wbase + i * K, tail_base)
        pltpu.sync_copy(out_v, out_hbm.at[pl.ds(obase, K)])
        return 0

    lax.fori_loop(0, nsteps, step, 0)


@jax.jit
def kernel(z_src, z_dst, edge_label_index):
    edge = edge_label_index.astype(jnp.int32)
    src_idx = edge[0]
    dst_idx = edge[1]
    f = functools.partial(
        pl.kernel,
        mesh=plsc.VectorSubcoreMesh(core_axis_name="c", subcore_axis_name="s"),
        out_type=jax.ShapeDtypeStruct((E,), jnp.float32),
        compiler_params=pltpu.CompilerParams(use_tc_tiling_on_sc=False,
                                             needs_layout_passes=False),
        scratch_types=[
            pltpu.VMEM((IDX_CAP,), jnp.int32),
            pltpu.VMEM((IDX_CAP,), jnp.int32),
            pltpu.VMEM((K, D), jnp.float32),
            pltpu.VMEM((K, D), jnp.float32),
            pltpu.VMEM((K,), jnp.float32),
            pltpu.SemaphoreType.DMA,
            pltpu.SemaphoreType.DMA,
        ],
    )(_dot_decoder_body)
    return f(src_idx, dst_idx, z_src, z_dst)


# speedup vs baseline: 4.1281x; 4.1281x over previous
import functools

import jax
import jax.numpy as jnp
from jax import lax
from jax.experimental import pallas as pl
from jax.experimental.pallas import tpu as pltpu
from jax.experimental.pallas import tpu_sc as plsc

E = 160000
D = 256
NW = 32
K = 80
MAIN_STEPS = 62
MAIN_E = NW * MAIN_STEPS * K
IDX_CAP = (MAIN_STEPS + 1) * K
LANES = 16
GROUPS = K // LANES


def _dot_decoder_body(src_hbm, dst_hbm, zsrc_hbm, zdst_hbm, out_hbm,
                      idx_s, idx_d, rows_s, rows_d, pbuf, out_v,
                      sem_s, sem_d):
    wid = lax.axis_index("s") * 2 + lax.axis_index("c")
    wbase = wid * MAIN_STEPS * K
    tail_base = MAIN_E + wid * K

    pltpu.sync_copy(src_hbm.at[pl.ds(wbase, MAIN_STEPS * K)],
                    idx_s.at[pl.ds(0, MAIN_STEPS * K)])
    pltpu.sync_copy(dst_hbm.at[pl.ds(wbase, MAIN_STEPS * K)],
                    idx_d.at[pl.ds(0, MAIN_STEPS * K)])

    @pl.when(wid < 16)
    def _stage_tail():
        pltpu.sync_copy(src_hbm.at[pl.ds(tail_base, K)],
                        idx_s.at[pl.ds(MAIN_STEPS * K, K)])
        pltpu.sync_copy(dst_hbm.at[pl.ds(tail_base, K)],
                        idx_d.at[pl.ds(MAIN_STEPS * K, K)])

    nsteps = jnp.where(wid < 16, MAIN_STEPS + 1, MAIN_STEPS)

    def step(i, _):
        cp_s = pltpu.async_copy(
            zsrc_hbm.at[idx_s.at[pl.ds(i * K, K)]], rows_s, sem_s)
        cp_d = pltpu.async_copy(
            zdst_hbm.at[idx_d.at[pl.ds(i * K, K)]], rows_d, sem_d)
        cp_s.wait()
        cp_d.wait()

        def row(r, _):
            prods = [rows_s[r, pl.ds(j * LANES, LANES)]
                     * rows_d[r, pl.ds(j * LANES, LANES)]
                     for j in range(D // LANES)]
            while len(prods) > 1:
                prods = [a + b for a, b in zip(prods[::2], prods[1::2])]
            pbuf[r, :] = prods[0]
            return 0

        lax.fori_loop(0, K, row, 0)

        for g in range(GROUPS):
            rowids = jnp.arange(LANES, dtype=jnp.int32) + g * LANES
            cols = [plsc.load_gather(
                        pbuf, [rowids, jnp.full((LANES,), j, jnp.int32)])
                    for j in range(LANES)]
            while len(cols) > 1:
                cols = [a + b for a, b in zip(cols[::2], cols[1::2])]
            out_v[pl.ds(g * LANES, LANES)] = cols[0]

        obase = jnp.where(i < MAIN_STEPS, wbase + i * K, tail_base)
        pltpu.sync_copy(out_v, out_hbm.at[pl.ds(obase, K)])
        return 0

    lax.fori_loop(0, nsteps, step, 0)


@jax.jit
def kernel(z_src, z_dst, edge_label_index):
    edge = edge_label_index.astype(jnp.int32)
    src_idx = edge[0]
    dst_idx = edge[1]
    f = functools.partial(
        pl.kernel,
        mesh=plsc.VectorSubcoreMesh(core_axis_name="c", subcore_axis_name="s"),
        out_type=jax.ShapeDtypeStruct((E,), jnp.float32),
        compiler_params=pltpu.CompilerParams(use_tc_tiling_on_sc=False,
                                             needs_layout_passes=False),
        scratch_types=[
            pltpu.VMEM((IDX_CAP,), jnp.int32),
            pltpu.VMEM((IDX_CAP,), jnp.int32),
            pltpu.VMEM((K, D), jnp.float32),
            pltpu.VMEM((K, D), jnp.float32),
            pltpu.VMEM((K, LANES), jnp.float32),
            pltpu.VMEM((K,), jnp.float32),
            pltpu.SemaphoreType.DMA,
            pltpu.SemaphoreType.DMA,
        ],
    )(_dot_decoder_body)
    return f(src_idx, dst_idx, z_src, z_dst)


# speedup vs baseline: 6.9226x; 1.6769x over previous
import functools

import jax
import jax.numpy as jnp
from jax import lax
from jax.experimental import pallas as pl
from jax.experimental.pallas import tpu as pltpu
from jax.experimental.pallas import tpu_sc as plsc

E = 160000
D = 256
NW = 32
K = 80
MAIN_STEPS = 62
MAIN_E = NW * MAIN_STEPS * K
IDX_CAP = (MAIN_STEPS + 1) * K
LANES = 16
GROUPS = K // LANES
HALF_ITERS = MAIN_STEPS // 2


def _dot_decoder_body(src_hbm, dst_hbm, zsrc_hbm, zdst_hbm, out_hbm,
                      idx_s, idx_d, rows_s, rows_d, pbuf, out_all,
                      sem_s0, sem_s1, sem_d0, sem_d1):
    wid = lax.axis_index("s") * 2 + lax.axis_index("c")
    wbase = wid * MAIN_STEPS * K
    tail_base = MAIN_E + wid * K
    sems = ((sem_s0, sem_d0), (sem_s1, sem_d1))

    pltpu.sync_copy(src_hbm.at[pl.ds(wbase, MAIN_STEPS * K)],
                    idx_s.at[pl.ds(0, MAIN_STEPS * K)])
    pltpu.sync_copy(dst_hbm.at[pl.ds(wbase, MAIN_STEPS * K)],
                    idx_d.at[pl.ds(0, MAIN_STEPS * K)])

    @pl.when(wid < 16)
    def _stage_tail():
        pltpu.sync_copy(src_hbm.at[pl.ds(tail_base, K)],
                        idx_s.at[pl.ds(MAIN_STEPS * K, K)])
        pltpu.sync_copy(dst_hbm.at[pl.ds(tail_base, K)],
                        idx_d.at[pl.ds(MAIN_STEPS * K, K)])

    nsteps = jnp.where(wid < 16, MAIN_STEPS + 1, MAIN_STEPS)

    def start(s, b):
        pltpu.async_copy(zsrc_hbm.at[idx_s.at[pl.ds(s * K, K)]],
                         rows_s.at[b], sems[b][0])
        pltpu.async_copy(zdst_hbm.at[idx_d.at[pl.ds(s * K, K)]],
                         rows_d.at[b], sems[b][1])

    def wait(s, b):
        pltpu.make_async_copy(zsrc_hbm.at[idx_s.at[pl.ds(s * K, K)]],
                              rows_s.at[b], sems[b][0]).wait()
        pltpu.make_async_copy(zdst_hbm.at[idx_d.at[pl.ds(s * K, K)]],
                              rows_d.at[b], sems[b][1]).wait()

    def compute(s, b):
        rs = rows_s.at[b]
        rd = rows_d.at[b]

        def row(r, _):
            prods = [rs[r, pl.ds(j * LANES, LANES)]
                     * rd[r, pl.ds(j * LANES, LANES)]
                     for j in range(D // LANES)]
            while len(prods) > 1:
                prods = [a + b_ for a, b_ in zip(prods[::2], prods[1::2])]
            pbuf[r, :] = prods[0]
            return 0

        lax.fori_loop(0, K, row, 0)

        for g in range(GROUPS):
            rowids = jnp.arange(LANES, dtype=jnp.int32) + g * LANES
            cols = [plsc.load_gather(
                        pbuf, [rowids, jnp.full((LANES,), j, jnp.int32)])
                    for j in range(LANES)]
            while len(cols) > 1:
                cols = [a + b_ for a, b_ in zip(cols[::2], cols[1::2])]
            out_all[pl.ds(s * K + g * LANES, LANES)] = cols[0]

    start(0, 0)

    def pair(it, _):
        s0 = it * 2
        start(s0 + 1, 1)
        wait(s0, 0)
        compute(s0, 0)

        @pl.when(s0 + 2 < nsteps)
        def _start_next():
            start(s0 + 2, 0)

        wait(s0 + 1, 1)
        compute(s0 + 1, 1)
        return 0

    lax.fori_loop(0, HALF_ITERS, pair, 0)

    @pl.when(wid < 16)
    def _tail_step():
        wait(MAIN_STEPS, 0)
        compute(MAIN_STEPS, 0)

    pltpu.sync_copy(out_all.at[pl.ds(0, MAIN_STEPS * K)],
                    out_hbm.at[pl.ds(wbase, MAIN_STEPS * K)])

    @pl.when(wid < 16)
    def _store_tail():
        pltpu.sync_copy(out_all.at[pl.ds(MAIN_STEPS * K, K)],
                        out_hbm.at[pl.ds(tail_base, K)])


@jax.jit
def kernel(z_src, z_dst, edge_label_index):
    edge = edge_label_index.astype(jnp.int32)
    src_idx = edge[0]
    dst_idx = edge[1]
    f = functools.partial(
        pl.kernel,
        mesh=plsc.VectorSubcoreMesh(core_axis_name="c", subcore_axis_name="s"),
        out_type=jax.ShapeDtypeStruct((E,), jnp.float32),
        compiler_params=pltpu.CompilerParams(use_tc_tiling_on_sc=False,
                                             needs_layout_passes=False),
        scratch_types=[
            pltpu.VMEM((IDX_CAP,), jnp.int32),
            pltpu.VMEM((IDX_CAP,), jnp.int32),
            pltpu.VMEM((2, K, D), jnp.float32),
            pltpu.VMEM((2, K, D), jnp.float32),
            pltpu.VMEM((K, LANES), jnp.float32),
            pltpu.VMEM((IDX_CAP,), jnp.float32),
            pltpu.SemaphoreType.DMA,
            pltpu.SemaphoreType.DMA,
            pltpu.SemaphoreType.DMA,
            pltpu.SemaphoreType.DMA,
        ],
    )(_dot_decoder_body)
    return f(src_idx, dst_idx, z_src, z_dst)
